# Initial kernel scaffold; baseline (speedup 1.0000x reference)
import functools

import jax
import jax.numpy as jnp
from jax import lax
from jax.experimental import pallas as pl
from jax.experimental.pallas import tpu as pltpu
from jax.experimental.pallas import tpu_sc as plsc

NS = 16
C = 128


def _hop_kernel(N_FULL, D, NCH):
    RPT = N_FULL // NS
    mesh = plsc.VectorSubcoreMesh(
        core_axis_name="c", subcore_axis_name="s", num_cores=1)

    @functools.partial(
        pl.kernel,
        out_type=jax.ShapeDtypeStruct((N_FULL, D), jnp.float32),
        mesh=mesh,
        scratch_types=[
            pltpu.VMEM((NCH, C), jnp.int32),
            pltpu.VMEM((NCH, C), jnp.int32),
            pltpu.VMEM((C, D), jnp.float32),
            pltpu.VMEM_SHARED((N_FULL, D), jnp.float32),
            pltpu.SemaphoreType.DMA,
        ],
    )
    def hop(h_hbm, src_hbm, dst_hbm, zeros_hbm, out_hbm,
            src_v, dst_v, rows_v, acc, sem):
        s = lax.axis_index("s")
        base = s * RPT

        pltpu.sync_copy(src_hbm.at[s], src_v)
        pltpu.sync_copy(dst_hbm.at[s], dst_v)

        nz = RPT // C
        for z in range(nz):
            pltpu.sync_copy(zeros_hbm, acc.at[pl.ds(base + z * C, C)])
        rem = RPT - nz * C
        if rem:
            pltpu.sync_copy(zeros_hbm.at[pl.ds(0, rem)],
                            acc.at[pl.ds(base + nz * C, rem)])
        plsc.subcore_barrier()

        def chunk(j, carry):
            pltpu.async_copy(h_hbm.at[src_v.at[j]], rows_v, sem).wait()
            pltpu.sync_copy(rows_v, acc.at[dst_v.at[j]], add=True)
            return carry
        lax.fori_loop(0, NCH, chunk, 0)
        plsc.subcore_barrier()

        for z in range(nz):
            pltpu.sync_copy(acc.at[pl.ds(base + z * C, C)],
                            out_hbm.at[pl.ds(base + z * C, C)])
        if rem:
            pltpu.sync_copy(acc.at[pl.ds(base + nz * C, rem)],
                            out_hbm.at[pl.ds(base + nz * C, rem)])

    return hop


def _att_sum_kernel(hs_ref, att_ref, out_ref):
    acc = att_ref[0] * hs_ref[0]
    for k in range(1, hs_ref.shape[0]):
        acc = acc + att_ref[k] * hs_ref[k]
    out_ref[...] = acc


def kernel(x, edge_index, att):
    N, D = x.shape
    E = edge_index.shape[1]
    K = att.shape[0] - 1

    N_FULL = ((N + 2 * NS - 1) // (2 * NS)) * (2 * NS)
    if N_FULL == N:
        N_FULL = N + 2 * NS
    per_w = ((E + NS * C - 1) // (NS * C)) * C
    E_pad = per_w * NS
    NCH = per_w // C

    src = jnp.concatenate(
        [edge_index[0], jnp.zeros((E_pad - E,), jnp.int32)]).reshape(NS, NCH, C)
    dst = jnp.concatenate(
        [edge_index[1], jnp.full((E_pad - E,), N, jnp.int32)]).reshape(NS, NCH, C)

    x_full = jnp.pad(x, ((0, N_FULL - N), (0, 0)))
    zeros = jnp.zeros((C, D), jnp.float32)

    hop = _hop_kernel(N_FULL, D, NCH)
    hs = [x_full]
    h = x_full
    for _ in range(K):
        h = hop(h, src, dst, zeros)
        hs.append(h)
    hsf = jnp.stack(hs)

    BR = N_FULL // NS
    out_full = pl.pallas_call(
        _att_sum_kernel,
        grid=(NS,),
        in_specs=[
            pl.BlockSpec((K + 1, BR, D), lambda i: (0, i, 0)),
            pl.BlockSpec(memory_space=pltpu.SMEM),
        ],
        out_specs=pl.BlockSpec((BR, D), lambda i: (i, 0)),
        out_shape=jax.ShapeDtypeStruct((N_FULL, D), jnp.float32),
    )(hsf, att)
    return out_full[:N]


# speedup vs baseline: 2.5077x; 2.5077x over previous
import functools

import jax
import jax.numpy as jnp
from jax import lax
from jax.experimental import pallas as pl
from jax.experimental.pallas import tpu as pltpu
from jax.experimental.pallas import tpu_sc as plsc

NS = 16
C = 128


def _prop_kernel(N_FULL, D, NCH, K):
    RPT = N_FULL // NS
    nz, rem = RPT // C, RPT % C
    mesh = plsc.VectorSubcoreMesh(
        core_axis_name="c", subcore_axis_name="s", num_cores=1)

    @functools.partial(
        pl.kernel,
        out_type=jax.ShapeDtypeStruct((K, N_FULL, D), jnp.float32),
        mesh=mesh,
        scratch_types=[
            pltpu.VMEM((1, C), jnp.int32),
            pltpu.VMEM((1, C), jnp.int32),
            pltpu.VMEM((C, D), jnp.float32),
            pltpu.VMEM_SHARED((N_FULL, D), jnp.float32),
            pltpu.SemaphoreType.DMA,
        ],
    )
    def prop(x_hbm, src_hbm, dst_hbm, zeros_hbm, out_hbm,
             src_v, dst_v, rows_v, acc, sem):
        s = lax.axis_index("s")
        base = s * RPT

        for k in range(K):
            for z in range(nz):
                pltpu.sync_copy(zeros_hbm, acc.at[pl.ds(base + z * C, C)])
            if rem:
                pltpu.sync_copy(zeros_hbm.at[pl.ds(0, rem)],
                                acc.at[pl.ds(base + nz * C, rem)])
            plsc.subcore_barrier()

            h_ref = x_hbm if k == 0 else out_hbm.at[k - 1]

            def chunk(j, carry):
                pltpu.sync_copy(src_hbm.at[s].at[j], src_v.at[0])
                pltpu.sync_copy(dst_hbm.at[s].at[j], dst_v.at[0])
                pltpu.async_copy(h_ref.at[src_v.at[0]], rows_v, sem).wait()
                pltpu.sync_copy(rows_v, acc.at[dst_v.at[0]], add=True)
                return carry
            lax.fori_loop(0, NCH, chunk, 0)
            plsc.subcore_barrier()

            for z in range(nz):
                pltpu.sync_copy(acc.at[pl.ds(base + z * C, C)],
                                out_hbm.at[k].at[pl.ds(base + z * C, C)])
            if rem:
                pltpu.sync_copy(acc.at[pl.ds(base + nz * C, rem)],
                                out_hbm.at[k].at[pl.ds(base + nz * C, rem)])

    return prop


def _att_sum_kernel(x_ref, hs_ref, att_ref, out_ref):
    acc = att_ref[0] * x_ref[...]
    for k in range(hs_ref.shape[0]):
        acc = acc + att_ref[k + 1] * hs_ref[k]
    out_ref[...] = acc


def kernel(x, edge_index, att):
    N, D = x.shape
    E = edge_index.shape[1]
    K = att.shape[0] - 1

    N_FULL = ((N + C) // C) * C
    per_w = ((E + NS * C - 1) // (NS * C)) * C
    E_pad = per_w * NS
    NCH = per_w // C

    src = jnp.concatenate(
        [edge_index[0], jnp.zeros((E_pad - E,), jnp.int32)]).reshape(NS, NCH, C)
    dst = jnp.concatenate(
        [edge_index[1], jnp.full((E_pad - E,), N, jnp.int32)]).reshape(NS, NCH, C)

    x_full = jnp.pad(x, ((0, N_FULL - N), (0, 0)))
    zeros = jnp.zeros((C, D), jnp.float32)

    hs = _prop_kernel(N_FULL, D, NCH, K)(x_full, src, dst, zeros)

    BR = 32
    out_full = pl.pallas_call(
        _att_sum_kernel,
        grid=(N_FULL // BR,),
        in_specs=[
            pl.BlockSpec((BR, D), lambda i: (i, 0)),
            pl.BlockSpec((K, BR, D), lambda i: (0, i, 0)),
            pl.BlockSpec(memory_space=pltpu.SMEM),
        ],
        out_specs=pl.BlockSpec((BR, D), lambda i: (i, 0)),
        out_shape=jax.ShapeDtypeStruct((N_FULL, D), jnp.float32),
    )(x_full, hs, att)
    return out_full[:N]


# speedup vs baseline: 2.6238x; 1.0463x over previous
import functools

import jax
import jax.numpy as jnp
from jax import lax
from jax.experimental import pallas as pl
from jax.experimental.pallas import tpu as pltpu
from jax.experimental.pallas import tpu_sc as plsc

NS = 16
C = 128


def _prop_kernel(N_FULL, D, NCH, K):
    RPT = N_FULL // NS
    nz, rem = RPT // C, RPT % C
    mesh = plsc.VectorSubcoreMesh(
        core_axis_name="c", subcore_axis_name="s", num_cores=1)

    @functools.partial(
        pl.kernel,
        out_type=jax.ShapeDtypeStruct((K, N_FULL, D), jnp.float32),
        mesh=mesh,
        scratch_types=[
            pltpu.VMEM((2, C), jnp.int32),
            pltpu.VMEM((2, C), jnp.int32),
            pltpu.VMEM((2, C, D), jnp.float32),
            pltpu.VMEM_SHARED((N_FULL, D), jnp.float32),
            pltpu.SemaphoreType.DMA,
            pltpu.SemaphoreType.DMA,
        ],
    )
    def prop(x_hbm, src_hbm, dst_hbm, zeros_hbm, out_hbm,
             src_v, dst_v, rows_v, acc, sem0, sem1):
        s = lax.axis_index("s")
        base = s * RPT
        sems = (sem0, sem1)

        def stage(j, b):
            pltpu.sync_copy(src_hbm.at[s].at[j], src_v.at[b])
            pltpu.sync_copy(dst_hbm.at[s].at[j], dst_v.at[b])

        for k in range(K):
            for z in range(nz):
                pltpu.sync_copy(zeros_hbm, acc.at[pl.ds(base + z * C, C)])
            if rem:
                pltpu.sync_copy(zeros_hbm.at[pl.ds(0, rem)],
                                acc.at[pl.ds(base + nz * C, rem)])
            plsc.subcore_barrier()

            h_ref = x_hbm if k == 0 else out_hbm.at[k - 1]

            def gather(j, b):
                pltpu.async_copy(h_ref.at[src_v.at[b]], rows_v.at[b], sems[b])

            def gwait(b):
                pltpu.make_async_copy(
                    h_ref.at[src_v.at[b]], rows_v.at[b], sems[b]).wait()

            def scatter(b):
                pltpu.sync_copy(rows_v.at[b], acc.at[dst_v.at[b]], add=True)

            stage(0, 0)
            gather(0, 0)

            def pair(j2, carry):
                j = 2 * j2
                stage(j + 1, 1)
                gather(j + 1, 1)
                gwait(0)
                scatter(0)
                stage(j + 2, 0)
                gather(j + 2, 0)
                gwait(1)
                scatter(1)
                return carry
            lax.fori_loop(0, NCH // 2, pair, 0)
            gwait(0)
            plsc.subcore_barrier()

            for z in range(nz):
                pltpu.sync_copy(acc.at[pl.ds(base + z * C, C)],
                                out_hbm.at[k].at[pl.ds(base + z * C, C)])
            if rem:
                pltpu.sync_copy(acc.at[pl.ds(base + nz * C, rem)],
                                out_hbm.at[k].at[pl.ds(base + nz * C, rem)])

    return prop


def _att_sum_kernel(x_ref, hs_ref, att_ref, out_ref):
    acc = att_ref[0] * x_ref[...]
    for k in range(hs_ref.shape[0]):
        acc = acc + att_ref[k + 1] * hs_ref[k]
    out_ref[...] = acc


def kernel(x, edge_index, att):
    N, D = x.shape
    E = edge_index.shape[1]
    K = att.shape[0] - 1

    N_FULL = ((N + C) // C) * C
    per_w = ((E + NS * 2 * C - 1) // (NS * 2 * C)) * (2 * C)
    E_pad = per_w * NS
    NCH = per_w // C

    src = jnp.concatenate(
        [edge_index[0], jnp.zeros((E_pad - E,), jnp.int32)]).reshape(NS, NCH, C)
    dst = jnp.concatenate(
        [edge_index[1], jnp.full((E_pad - E,), N, jnp.int32)]).reshape(NS, NCH, C)
    src = jnp.concatenate([src, jnp.zeros((NS, 1, C), jnp.int32)], axis=1)
    dst = jnp.concatenate([dst, jnp.full((NS, 1, C), N, jnp.int32)], axis=1)

    x_full = jnp.pad(x, ((0, N_FULL - N), (0, 0)))
    zeros = jnp.zeros((C, D), jnp.float32)

    hs = _prop_kernel(N_FULL, D, NCH, K)(x_full, src, dst, zeros)

    BR = 32
    out_full = pl.pallas_call(
        _att_sum_kernel,
        grid=(N_FULL // BR,),
        in_specs=[
            pl.BlockSpec((BR, D), lambda i: (i, 0)),
            pl.BlockSpec((K, BR, D), lambda i: (0, i, 0)),
            pl.BlockSpec(memory_space=pltpu.SMEM),
        ],
        out_specs=pl.BlockSpec((BR, D), lambda i: (i, 0)),
        out_shape=jax.ShapeDtypeStruct((N_FULL, D), jnp.float32),
    )(x_full, hs, att)
    return out_full[:N]
